# baseline (speedup 1.0000x reference)
import functools

import jax
import jax.numpy as jnp
from jax import lax
from jax.experimental import pallas as pl
from jax.experimental.pallas import tpu as pltpu
from jax.experimental.pallas import tpu_sc as plsc

_NC = 2
_NS = 16
_NW = _NC * _NS
_L = 16


def _node_scalar_body(x_ref, we1_ref, be1_ref, we2_ref, be2_ref,
                      wa1_ref, ba1_ref, wa2_ref, ba2_ref, out_ref):
    x = x_ref[...]
    h = jnp.maximum(
        jnp.dot(x, we1_ref[...], preferred_element_type=jnp.float32)
        + be1_ref[...], 0.0)
    e = (jnp.dot(h, we2_ref[...], preferred_element_type=jnp.float32)
         + be2_ref[...])
    a = jnp.maximum(
        jnp.dot(e, wa1_ref[...], preferred_element_type=jnp.float32)
        + ba1_ref[...], 0.0)
    out_ref[...] = (
        jnp.dot(a, wa2_ref[...], preferred_element_type=jnp.float32)
        + ba2_ref[...])


def _node_scalars(nfeat, W_e1, b_e1, W_e2, b_e2, W_a1, b_a1, W_a2, b_a2,
                  block_n):
    n, d = nfeat.shape
    h = W_e1.shape[1]
    grid = (n // block_n,)
    full = lambda i: (0, 0)
    return pl.pallas_call(
        _node_scalar_body,
        grid=grid,
        in_specs=[
            pl.BlockSpec((block_n, d), lambda i: (i, 0)),
            pl.BlockSpec((d, h), full),
            pl.BlockSpec((1, h), full),
            pl.BlockSpec((h, h), full),
            pl.BlockSpec((1, h), full),
            pl.BlockSpec((h, h), full),
            pl.BlockSpec((1, h), full),
            pl.BlockSpec((h, 1), full),
            pl.BlockSpec((1, 1), full),
        ],
        out_specs=pl.BlockSpec((block_n, 1), lambda i: (i, 0)),
        out_shape=jax.ShapeDtypeStruct((n, 1), jnp.float32),
    )(nfeat, W_e1, b_e1, W_e2, b_e2, W_a1, b_a1, W_a2, b_a2)


def _make_sc_edge_mean(n, e4, s4, e8, s8):
    e4w = e4 // _NW
    e8w = e8 // _NW
    mesh = plsc.VectorSubcoreMesh(
        core_axis_name="c", subcore_axis_name="s", num_cores=_NC)

    @functools.partial(
        pl.kernel,
        mesh=mesh,
        compiler_params=pltpu.CompilerParams(
            use_tc_tiling_on_sc=False, needs_layout_passes=False),
        out_type=jax.ShapeDtypeStruct((e4 + e8,), jnp.float32),
        scratch_types=[
            pltpu.VMEM((n,), jnp.float32),
            pltpu.VMEM((e4w * s4,), jnp.int32),
            pltpu.VMEM((e8w * s8,), jnp.int32),
            pltpu.VMEM((e4w,), jnp.float32),
            pltpu.VMEM((e8w,), jnp.float32),
            pltpu.VMEM_SHARED((n,), jnp.float32),
            pltpu.SemaphoreType.DMA,
            pltpu.SemaphoreType.DMA,
            pltpu.SemaphoreType.DMA,
        ],
    )
    def sc_edge_mean(v_hbm, he4_hbm, he8_hbm, out_hbm,
                     v_v, i4_v, i8_v, o4_v, o8_v, v_sh, sem0, sem1, sem2):
        sid = lax.axis_index("s")
        wid = sid * _NC + lax.axis_index("c")
        c4 = pltpu.async_copy(he4_hbm.at[wid], i4_v, sem1)
        c8 = pltpu.async_copy(he8_hbm.at[wid], i8_v, sem2)
        @pl.when(sid == 0)
        def _():
            pltpu.sync_copy(v_hbm, v_sh)
        plsc.subcore_barrier()
        cv = pltpu.async_copy(v_sh, v_v, sem0)
        cv.wait()
        c4.wait()
        c8.wait()
        lane = jnp.arange(_L, dtype=jnp.int32)
        for grp, i_v, o_v, ew in ((s4, i4_v, o4_v, e4w), (s8, i8_v, o8_v, e8w)):
            lane_g = lane * grp
            for c in range(ew // _L):
                acc = jnp.zeros((_L,), jnp.float32)
                for s in range(grp):
                    pos = lane_g + (c * _L * grp + s)
                    idx = plsc.load_gather(i_v, [pos])
                    acc = acc + plsc.load_gather(v_v, [idx])
                o_v[pl.ds(c * _L, _L)] = acc * (1.0 / grp)
        d4 = pltpu.async_copy(o4_v, out_hbm.at[pl.ds(wid * e4w, e4w)], sem1)
        d8 = pltpu.async_copy(o8_v, out_hbm.at[pl.ds(e4 + wid * e8w, e8w)], sem2)
        d4.wait()
        d8.wait()

    return sc_edge_mean


def kernel(nfeat, hedges_s4, hedges_s8, W_e1, b_e1, W_e2, b_e2,
           W_a1, b_a1, W_a2, b_a2):
    n, d = nfeat.shape
    e4, s4 = hedges_s4.shape
    e8, s8 = hedges_s8.shape

    block_n = 1000
    v2d = _node_scalars(nfeat, W_e1, b_e1.reshape(1, -1), W_e2,
                        b_e2.reshape(1, -1), W_a1, b_a1.reshape(1, -1),
                        W_a2, b_a2.reshape(1, 1), block_n)
    v = v2d.reshape(n)

    he4_w = hedges_s4.astype(jnp.int32).reshape(_NW, (e4 // _NW) * s4)
    he8_w = hedges_s8.astype(jnp.int32).reshape(_NW, (e8 // _NW) * s8)

    sc_fn = _make_sc_edge_mean(n, e4, s4, e8, s8)
    return sc_fn(v, he4_w, he8_w)


# speedup vs baseline: 1.0199x; 1.0199x over previous
import functools

import jax
import jax.numpy as jnp
from jax import lax
from jax.experimental import pallas as pl
from jax.experimental.pallas import tpu as pltpu
from jax.experimental.pallas import tpu_sc as plsc

_NC = 2
_NS = 16
_NW = _NC * _NS
_L = 16


def _node_scalar_body(x_ref, we1_ref, be1_ref, we2_ref, be2_ref,
                      wa1_ref, ba1_ref, wa2_ref, ba2_ref, out_ref):
    bf = jnp.bfloat16
    x = x_ref[...].astype(bf)
    h = jnp.maximum(
        jnp.dot(x, we1_ref[...].astype(bf), preferred_element_type=jnp.float32)
        + be1_ref[...], 0.0)
    e = (jnp.dot(h.astype(bf), we2_ref[...].astype(bf),
                 preferred_element_type=jnp.float32) + be2_ref[...])
    a = jnp.maximum(
        jnp.dot(e.astype(bf), wa1_ref[...].astype(bf),
                preferred_element_type=jnp.float32)
        + ba1_ref[...], 0.0)
    out_ref[...] = (
        jnp.dot(a, wa2_ref[...], preferred_element_type=jnp.float32)
        + ba2_ref[...])


def _node_scalars(nfeat, W_e1, b_e1, W_e2, b_e2, W_a1, b_a1, W_a2, b_a2,
                  block_n):
    n, d = nfeat.shape
    h = W_e1.shape[1]
    grid = (n // block_n,)
    full = lambda i: (0, 0)
    return pl.pallas_call(
        _node_scalar_body,
        grid=grid,
        in_specs=[
            pl.BlockSpec((block_n, d), lambda i: (i, 0)),
            pl.BlockSpec((d, h), full),
            pl.BlockSpec((1, h), full),
            pl.BlockSpec((h, h), full),
            pl.BlockSpec((1, h), full),
            pl.BlockSpec((h, h), full),
            pl.BlockSpec((1, h), full),
            pl.BlockSpec((h, 1), full),
            pl.BlockSpec((1, 1), full),
        ],
        out_specs=pl.BlockSpec((block_n, 1), lambda i: (i, 0)),
        out_shape=jax.ShapeDtypeStruct((n, 1), jnp.float32),
    )(nfeat, W_e1, b_e1, W_e2, b_e2, W_a1, b_a1, W_a2, b_a2)


def _make_sc_edge_mean(n, e4, s4, e8, s8):
    e4w = e4 // _NW
    e8w = e8 // _NW
    mesh = plsc.VectorSubcoreMesh(
        core_axis_name="c", subcore_axis_name="s", num_cores=_NC)

    @functools.partial(
        pl.kernel,
        mesh=mesh,
        compiler_params=pltpu.CompilerParams(
            use_tc_tiling_on_sc=False, needs_layout_passes=False),
        out_type=jax.ShapeDtypeStruct((e4 + e8,), jnp.float32),
        scratch_types=[
            pltpu.VMEM((n,), jnp.float32),
            pltpu.VMEM((e4w * s4,), jnp.int32),
            pltpu.VMEM((e8w * s8,), jnp.int32),
            pltpu.VMEM((e4w,), jnp.float32),
            pltpu.VMEM((e8w,), jnp.float32),
            pltpu.VMEM_SHARED((n,), jnp.float32),
            pltpu.SemaphoreType.DMA,
            pltpu.SemaphoreType.DMA,
            pltpu.SemaphoreType.DMA,
        ],
    )
    def sc_edge_mean(v_hbm, he4_hbm, he8_hbm, out_hbm,
                     v_v, i4_v, i8_v, o4_v, o8_v, v_sh, sem0, sem1, sem2):
        sid = lax.axis_index("s")
        wid = sid * _NC + lax.axis_index("c")
        c4 = pltpu.async_copy(he4_hbm.at[wid], i4_v, sem1)
        c8 = pltpu.async_copy(he8_hbm.at[wid], i8_v, sem2)
        @pl.when(sid == 0)
        def _():
            pltpu.sync_copy(v_hbm, v_sh)
        plsc.subcore_barrier()
        cv = pltpu.async_copy(v_sh, v_v, sem0)
        cv.wait()
        c4.wait()
        c8.wait()
        lane = jnp.arange(_L, dtype=jnp.int32)
        for grp, i_v, o_v, ew in ((s4, i4_v, o4_v, e4w), (s8, i8_v, o8_v, e8w)):
            lane_g = lane * grp
            for c in range(ew // _L):
                acc = jnp.zeros((_L,), jnp.float32)
                for s in range(grp):
                    pos = lane_g + (c * _L * grp + s)
                    idx = plsc.load_gather(i_v, [pos])
                    acc = acc + plsc.load_gather(v_v, [idx])
                o_v[pl.ds(c * _L, _L)] = acc * (1.0 / grp)
        d4 = pltpu.async_copy(o4_v, out_hbm.at[pl.ds(wid * e4w, e4w)], sem1)
        d8 = pltpu.async_copy(o8_v, out_hbm.at[pl.ds(e4 + wid * e8w, e8w)], sem2)
        d4.wait()
        d8.wait()

    return sc_edge_mean


def kernel(nfeat, hedges_s4, hedges_s8, W_e1, b_e1, W_e2, b_e2,
           W_a1, b_a1, W_a2, b_a2):
    n, d = nfeat.shape
    e4, s4 = hedges_s4.shape
    e8, s8 = hedges_s8.shape

    block_n = 1000
    v2d = _node_scalars(nfeat, W_e1, b_e1.reshape(1, -1), W_e2,
                        b_e2.reshape(1, -1), W_a1, b_a1.reshape(1, -1),
                        W_a2, b_a2.reshape(1, 1), block_n)
    v = v2d.reshape(n)

    he4_w = hedges_s4.astype(jnp.int32).reshape(_NW, (e4 // _NW) * s4)
    he8_w = hedges_s8.astype(jnp.int32).reshape(_NW, (e8 // _NW) * s8)

    sc_fn = _make_sc_edge_mean(n, e4, s4, e8, s8)
    return sc_fn(v, he4_w, he8_w)
